# Initial kernel scaffold; baseline (speedup 1.0000x reference)
import jax
import jax.numpy as jnp
from jax.experimental import pallas as pl


def kernel(labels, table):
    raise NotImplementedError("write your pallas kernel here")


# speedup vs baseline: 1.5409x; 1.5409x over previous
import functools

import jax
import jax.numpy as jnp
from jax import lax
from jax.experimental import pallas as pl
from jax.experimental.pallas import tpu as pltpu
from jax.experimental.pallas import tpu_sc as plsc

BATCH = 16384
HIDDEN = 1024
CHUNK = 64


@jax.jit
def _embed(labels, table):
    info = plsc.get_sparse_core_info()
    num_workers = info.num_cores * info.num_subcores
    b_per_w = BATCH // num_workers
    n_chunks = b_per_w // CHUNK

    mesh = plsc.VectorSubcoreMesh(core_axis_name="c", subcore_axis_name="s")

    @functools.partial(
        pl.kernel,
        mesh=mesh,
        out_type=jax.ShapeDtypeStruct((BATCH, HIDDEN), jnp.float32),
        scratch_types=[
            pltpu.VMEM((b_per_w,), jnp.int32),
            pltpu.VMEM((CHUNK, HIDDEN), jnp.float32),
            pltpu.SemaphoreType.DMA,
        ],
    )
    def k(labels_hbm, table_hbm, out_hbm, idx_v, rows_v, sem):
        wid = lax.axis_index("s") * info.num_cores + lax.axis_index("c")
        base = wid * b_per_w
        pltpu.sync_copy(labels_hbm.at[pl.ds(base, b_per_w)], idx_v)

        def body(i, carry):
            idx_chunk = idx_v.at[pl.ds(i * CHUNK, CHUNK)]
            pltpu.async_copy(table_hbm.at[idx_chunk], rows_v, sem).wait()
            pltpu.sync_copy(rows_v, out_hbm.at[pl.ds(base + i * CHUNK, CHUNK)])
            return carry

        lax.fori_loop(0, n_chunks, body, 0)

    return k(labels, table)


def kernel(labels, table):
    return _embed(labels, table)
